# baseline (speedup 1.0000x reference)
import functools

import jax
import jax.numpy as jnp
from jax import lax
from jax.experimental import pallas as pl
from jax.experimental.pallas import tpu as pltpu
from jax.experimental.pallas import tpu_sc as plsc

Q = 8192
T = 8192
D = 1024
BQ = 4096
BT = 256
NQ = Q // BQ
NT = T // BT
S = NT // 2
CR = 256
NCR = BQ // CR
LW = 128

_DN = (((1,), (1,)), ((), ()))


def _phase(s_ref, lm_ref, li_ref, dst, t_ref, src, ns_ref, row_base,
           nt_ref, cb, first):
    for r in range(NCR):
        sl = pl.ds(r * CR, CR)
        dst[sl, :] = lax.dot_general(
            s_ref[sl, :], t_ref[...], _DN,
            preferred_element_type=jnp.float32)
        ns = ns_ref[pl.ds(row_base + r * CR, CR), :]
        for c in range(BT // LW):
            cs = pl.ds(c * LW, LW)
            d = 1.0 - src[sl, cs] / (ns * nt_ref[:, cs])
            p = (BT // LW) * cb + c
            upd = d < lm_ref[sl, :]
            if c == 0 and first is not None:
                upd = jnp.logical_or(upd, first)
            lm_ref[sl, :] = jnp.where(upd, d, lm_ref[sl, :])
            li_ref[sl, :] = jnp.where(upd, p, li_ref[sl, :])


def _argmin_body(s_ref, ta_ref, tb_ref, ns_ref, ntp_ref, nta_ref,
                 lm_ref, li_ref, a_buf, b_buf, row_base):
    q = pl.program_id(0)
    qc = jnp.minimum(q, S - 1)

    _phase(s_ref, lm_ref, li_ref, a_buf, ta_ref, b_buf,
           ns_ref, row_base, ntp_ref, 2 * q - 1, None)

    _phase(s_ref, lm_ref, li_ref, b_buf, tb_ref, a_buf,
           ns_ref, row_base, nta_ref, 2 * qc, q == 0)


def _argmin_state(source, target, ns_col, nt_row, ib, interpret=False):
    def body(s_ref, ta_ref, tb_ref, ns_ref, ntp_ref, nta_ref,
             lm_ref, li_ref, a_buf, b_buf):
        _argmin_body(s_ref, ta_ref, tb_ref, ns_ref, ntp_ref, nta_ref,
                     lm_ref, li_ref, a_buf, b_buf, ib * BQ)

    return pl.pallas_call(
        body,
        grid=(S + 1,),
        in_specs=[
            pl.BlockSpec((BQ, D), lambda g: (ib, 0)),
            pl.BlockSpec((BT, D),
                         lambda g: (2 * jnp.minimum(g, S - 1), 0)),
            pl.BlockSpec((BT, D),
                         lambda g: (2 * jnp.minimum(g, S - 1) + 1, 0)),
            pl.BlockSpec((Q, 1), lambda g: (0, 0)),
            pl.BlockSpec((1, BT),
                         lambda g: (0, jnp.maximum(2 * g - 1, 0))),
            pl.BlockSpec((1, BT),
                         lambda g: (0, 2 * jnp.minimum(g, S - 1))),
        ],
        out_specs=[
            pl.BlockSpec((BQ, LW), lambda g: (0, 0)),
            pl.BlockSpec((BQ, LW), lambda g: (0, 0)),
        ],
        out_shape=[
            jax.ShapeDtypeStruct((BQ, LW), jnp.float32),
            jax.ShapeDtypeStruct((BQ, LW), jnp.int32),
        ],
        scratch_shapes=[
            pltpu.VMEM((BQ, BT), jnp.float32),
            pltpu.VMEM((BQ, BT), jnp.float32),
        ],
        compiler_params=pltpu.CompilerParams(
            dimension_semantics=("arbitrary",)),
        interpret=interpret,
    )(source, target, target, ns_col, nt_row, nt_row)


def _finalize_body(lm_ref, li_ref, idx_ref):
    lm = lm_ref[...]
    m = jnp.min(lm, axis=1, keepdims=True)
    col = li_ref[...] * LW + lax.broadcasted_iota(jnp.int32, (BQ, LW), 1)
    idx_ref[...] = jnp.min(
        jnp.where(lm == m, col, T), axis=1, keepdims=True)


def _finalize(lm, li, interpret=False):
    return pl.pallas_call(
        _finalize_body,
        grid=(1,),
        in_specs=[
            pl.BlockSpec((BQ, LW), lambda i: (0, 0)),
            pl.BlockSpec((BQ, LW), lambda i: (0, 0)),
        ],
        out_specs=pl.BlockSpec((BQ, 1), lambda i: (0, 0)),
        out_shape=jax.ShapeDtypeStruct((BQ, 1), jnp.int32),
        interpret=interpret,
    )(lm, li)


_NC = 2
_NS = 16
_NW = _NC * _NS
_BPW = BQ // _NW
_CH = 64
_NCH = _BPW // _CH


def _gather_body(table_hbm, idx_hbm, out_hbm, idx_v, rows_v, sem):
    wid = lax.axis_index("s") * _NC + lax.axis_index("c")
    base = wid * _BPW
    for c in range(_NCH):
        off = base + c * _CH
        pltpu.sync_copy(idx_hbm.at[pl.ds(off, _CH)], idx_v)
        pltpu.async_copy(table_hbm.at[idx_v], rows_v, sem).wait()
        pltpu.sync_copy(rows_v, out_hbm.at[pl.ds(off, _CH)])


def _sc_gather(table, idx):
    k = functools.partial(
        pl.kernel,
        mesh=plsc.VectorSubcoreMesh(
            core_axis_name="c", subcore_axis_name="s",
            num_cores=_NC, num_subcores=_NS),
        out_type=jax.ShapeDtypeStruct((BQ, D), jnp.float32),
        scratch_types=[
            pltpu.VMEM((_CH,), jnp.int32),
            pltpu.VMEM((_CH, D), jnp.float32),
            pltpu.SemaphoreType.DMA,
        ],
    )(_gather_body)
    return k(table, idx)


def kernel(source_features, target_features):
    source_norms = jnp.linalg.norm(source_features, axis=-1)
    matching_norms = jnp.linalg.norm(target_features, axis=-1)
    ns_col = source_norms.reshape(Q, 1)
    nt_row = matching_norms.reshape(1, T)
    outs = []
    for ib in range(NQ):
        lm, li = _argmin_state(
            source_features, target_features, ns_col, nt_row, ib)
        idx = _finalize(lm, li).reshape(BQ)
        outs.append(_sc_gather(target_features, idx))
    return jnp.concatenate(outs, axis=0)


# speedup vs baseline: 1.1006x; 1.1006x over previous
import functools

import jax
import jax.numpy as jnp
from jax import lax
from jax.experimental import pallas as pl
from jax.experimental.pallas import tpu as pltpu
from jax.experimental.pallas import tpu_sc as plsc

Q = 8192
T = 8192
D = 1024
BQ = 4096
BT = 256
NQ = Q // BQ
NT = T // BT
S = NT // 2
CR = 256
NCR = BQ // CR
LW = 128

_DN = (((1,), (1,)), ((), ()))


def _phase(s_ref, lm_ref, li_ref, dst, t_ref, src, ns_ref, row_base,
           nt_ref, cb, first):
    for r in range(NCR):
        sl = pl.ds(r * CR, CR)
        dst[sl, :] = lax.dot_general(
            s_ref[sl, :], t_ref[...], _DN,
            preferred_element_type=jnp.float32)
        ns = ns_ref[pl.ds(row_base + r * CR, CR), :]
        for c in range(BT // LW):
            cs = pl.ds(c * LW, LW)
            d = 1.0 - src[sl, cs] / (ns * nt_ref[:, cs])
            p = (BT // LW) * cb + c
            upd = d < lm_ref[sl, :]
            if c == 0 and first is not None:
                upd = jnp.logical_or(upd, first)
            lm_ref[sl, :] = jnp.where(upd, d, lm_ref[sl, :])
            li_ref[sl, :] = jnp.where(upd, p, li_ref[sl, :])


def _argmin_body(s_ref, ta_ref, tb_ref, ns_ref, ntp_ref, nta_ref,
                 lm_ref, li_ref, a_buf, b_buf):
    g = pl.program_id(0)
    q = g % (S + 1)
    qc = jnp.minimum(q, S - 1)
    row_base = (g // (S + 1)) * BQ

    _phase(s_ref, lm_ref, li_ref, a_buf, ta_ref, b_buf,
           ns_ref, row_base, ntp_ref, 2 * q - 1, None)

    _phase(s_ref, lm_ref, li_ref, b_buf, tb_ref, a_buf,
           ns_ref, row_base, nta_ref, 2 * qc, q == 0)


def _argmin_state(source, target, ns_col, nt_row, interpret=False):
    sp1 = S + 1
    grid = (NQ * sp1,)
    return pl.pallas_call(
        _argmin_body,
        grid=grid,
        in_specs=[
            pl.BlockSpec((BQ, D), lambda g: (g // (S + 1), 0)),
            pl.BlockSpec((BT, D),
                         lambda g: (2 * jnp.minimum(g % (S + 1), S - 1), 0)),
            pl.BlockSpec((BT, D),
                         lambda g: (2 * jnp.minimum(g % (S + 1), S - 1) + 1, 0)),
            pl.BlockSpec((Q, 1), lambda g: (0, 0)),
            pl.BlockSpec((1, BT),
                         lambda g: (0, jnp.maximum(2 * (g % (S + 1)) - 1, 0))),
            pl.BlockSpec((1, BT),
                         lambda g: (0, 2 * jnp.minimum(g % (S + 1), S - 1))),
        ],
        out_specs=[
            pl.BlockSpec((BQ, LW), lambda g: (g // (S + 1), 0)),
            pl.BlockSpec((BQ, LW), lambda g: (g // (S + 1), 0)),
        ],
        out_shape=[
            jax.ShapeDtypeStruct((Q, LW), jnp.float32),
            jax.ShapeDtypeStruct((Q, LW), jnp.int32),
        ],
        scratch_shapes=[
            pltpu.VMEM((BQ, BT), jnp.float32),
            pltpu.VMEM((BQ, BT), jnp.float32),
        ],
        compiler_params=pltpu.CompilerParams(
            dimension_semantics=("arbitrary",)),
        interpret=interpret,
    )(source, target, target, ns_col, nt_row, nt_row)


def _finalize_body(lm_ref, li_ref, idx_ref):
    lm = lm_ref[...]
    m = jnp.min(lm, axis=1, keepdims=True)
    col = li_ref[...] * LW + lax.broadcasted_iota(jnp.int32, (BQ, LW), 1)
    idx_ref[...] = jnp.min(
        jnp.where(lm == m, col, T), axis=1, keepdims=True)


def _finalize(lm, li, interpret=False):
    return pl.pallas_call(
        _finalize_body,
        grid=(NQ,),
        in_specs=[
            pl.BlockSpec((BQ, LW), lambda i: (i, 0)),
            pl.BlockSpec((BQ, LW), lambda i: (i, 0)),
        ],
        out_specs=pl.BlockSpec((BQ, 1), lambda i: (i, 0)),
        out_shape=jax.ShapeDtypeStruct((Q, 1), jnp.int32),
        interpret=interpret,
    )(lm, li)


_NC = 2
_NS = 16
_NW = _NC * _NS
_BPW = Q // _NW
_CH = 32
_NCH = _BPW // _CH


def _gather_body(table_hbm, idx_hbm, out_hbm, idx_v, r0, r1,
                 g0, g1, w0, w1):
    wid = lax.axis_index("s") * _NC + lax.axis_index("c")
    base = wid * _BPW
    pltpu.sync_copy(idx_hbm.at[pl.ds(base, _BPW)], idx_v)
    bufs = (r0, r1)
    gsems = (g0, g1)
    wsems = (w0, w1)
    gcp = [None, None]
    wcp = [None, None]
    gcp[0] = pltpu.async_copy(
        table_hbm.at[idx_v.at[pl.ds(0, _CH)]], bufs[0], gsems[0])
    for c in range(_NCH):
        b = c % 2
        nb = (c + 1) % 2
        if c + 1 < _NCH:
            if wcp[nb] is not None:
                wcp[nb].wait()
            gcp[nb] = pltpu.async_copy(
                table_hbm.at[idx_v.at[pl.ds((c + 1) * _CH, _CH)]],
                bufs[nb], gsems[nb])
        gcp[b].wait()
        wcp[b] = pltpu.async_copy(
            bufs[b], out_hbm.at[pl.ds(base + c * _CH, _CH)], wsems[b])
    for w in wcp:
        w.wait()


def _sc_gather(table, idx):
    k = functools.partial(
        pl.kernel,
        mesh=plsc.VectorSubcoreMesh(
            core_axis_name="c", subcore_axis_name="s",
            num_cores=_NC, num_subcores=_NS),
        out_type=jax.ShapeDtypeStruct((Q, D), jnp.float32),
        scratch_types=[
            pltpu.VMEM((_BPW,), jnp.int32),
            pltpu.VMEM((_CH, D), jnp.float32),
            pltpu.VMEM((_CH, D), jnp.float32),
            pltpu.SemaphoreType.DMA,
            pltpu.SemaphoreType.DMA,
            pltpu.SemaphoreType.DMA,
            pltpu.SemaphoreType.DMA,
        ],
    )(_gather_body)
    return k(table, idx)


def kernel(source_features, target_features):
    source_norms = jnp.linalg.norm(source_features, axis=-1)
    matching_norms = jnp.linalg.norm(target_features, axis=-1)
    lm, li = _argmin_state(
        source_features, target_features,
        source_norms.reshape(Q, 1), matching_norms.reshape(1, T))
    idx = _finalize(lm, li).reshape(Q)
    return _sc_gather(target_features, idx)


# speedup vs baseline: 1.1014x; 1.0007x over previous
import functools

import jax
import jax.numpy as jnp
from jax import lax
from jax.experimental import pallas as pl
from jax.experimental.pallas import tpu as pltpu
from jax.experimental.pallas import tpu_sc as plsc

Q = 8192
T = 8192
D = 1024
BQ = 4096
BT = 256
NQ = Q // BQ
NT = T // BT
S = NT // 2
CR = 256
NCR = BQ // CR
LW = 128

_DN = (((1,), (1,)), ((), ()))


def _phase(s_ref, lm_ref, li_ref, dst, t_ref, src, ns_ref, row_base,
           nt_ref, cb, first):
    for r in range(NCR):
        sl = pl.ds(r * CR, CR)
        dst[sl, :] = lax.dot_general(
            s_ref[sl, :], t_ref[...], _DN,
            preferred_element_type=jnp.float32)
        ns = ns_ref[pl.ds(row_base + r * CR, CR), :]
        for c in range(BT // LW):
            cs = pl.ds(c * LW, LW)
            d = 1.0 - src[sl, cs] / (ns * nt_ref[:, cs])
            p = (BT // LW) * cb + c
            upd = d < lm_ref[sl, :]
            if c == 0 and first is not None:
                upd = jnp.logical_or(upd, first)
            lm_ref[sl, :] = jnp.where(upd, d, lm_ref[sl, :])
            li_ref[sl, :] = jnp.where(upd, p, li_ref[sl, :])


def _argmin_body(s_ref, ta_ref, tb_ref, ns_ref, ntp_ref, nta_ref,
                 lm_ref, li_ref, a_buf, b_buf):
    g = pl.program_id(0)
    q = g % (S + 1)
    qc = jnp.minimum(q, S - 1)
    row_base = (g // (S + 1)) * BQ

    _phase(s_ref, lm_ref, li_ref, a_buf, ta_ref, b_buf,
           ns_ref, row_base, ntp_ref, 2 * q - 1, None)

    _phase(s_ref, lm_ref, li_ref, b_buf, tb_ref, a_buf,
           ns_ref, row_base, nta_ref, 2 * qc, q == 0)


def _argmin_state(source, target, ns_col, nt_row):
    sp1 = S + 1
    grid = (NQ * sp1,)
    return pl.pallas_call(
        _argmin_body,
        grid=grid,
        in_specs=[
            pl.BlockSpec((BQ, D), lambda g: (g // (S + 1), 0)),
            pl.BlockSpec((BT, D),
                         lambda g: (2 * jnp.minimum(g % (S + 1), S - 1), 0)),
            pl.BlockSpec((BT, D),
                         lambda g: (2 * jnp.minimum(g % (S + 1), S - 1) + 1, 0)),
            pl.BlockSpec((Q, 1), lambda g: (0, 0)),
            pl.BlockSpec((1, BT),
                         lambda g: (0, jnp.maximum(2 * (g % (S + 1)) - 1, 0))),
            pl.BlockSpec((1, BT),
                         lambda g: (0, 2 * jnp.minimum(g % (S + 1), S - 1))),
        ],
        out_specs=[
            pl.BlockSpec((BQ, LW), lambda g: (g // (S + 1), 0)),
            pl.BlockSpec((BQ, LW), lambda g: (g // (S + 1), 0)),
        ],
        out_shape=[
            jax.ShapeDtypeStruct((Q, LW), jnp.float32),
            jax.ShapeDtypeStruct((Q, LW), jnp.int32),
        ],
        scratch_shapes=[
            pltpu.VMEM((BQ, BT), jnp.float32),
            pltpu.VMEM((BQ, BT), jnp.float32),
        ],
        compiler_params=pltpu.CompilerParams(
            dimension_semantics=("arbitrary",)),

    )(source, target, target, ns_col, nt_row, nt_row)


def _finalize_body(lm_ref, li_ref, idx_ref):
    lm = lm_ref[...]
    m = jnp.min(lm, axis=1, keepdims=True)
    col = li_ref[...] * LW + lax.broadcasted_iota(jnp.int32, (BQ, LW), 1)
    idx_ref[...] = jnp.min(
        jnp.where(lm == m, col, T), axis=1, keepdims=True)


def _finalize(lm, li):
    return pl.pallas_call(
        _finalize_body,
        grid=(NQ,),
        in_specs=[
            pl.BlockSpec((BQ, LW), lambda i: (i, 0)),
            pl.BlockSpec((BQ, LW), lambda i: (i, 0)),
        ],
        out_specs=pl.BlockSpec((BQ, 1), lambda i: (i, 0)),
        out_shape=jax.ShapeDtypeStruct((Q, 1), jnp.int32),

    )(lm, li)


_NC = 2
_NS = 16
_NW = _NC * _NS
_BPW = Q // _NW
_CH = 32
_NCH = _BPW // _CH


def _gather_body(table_hbm, idx_hbm, out_hbm, idx_v, r0, r1,
                 g0, g1, w0, w1):
    wid = lax.axis_index("s") * _NC + lax.axis_index("c")
    base = wid * _BPW
    pltpu.sync_copy(idx_hbm.at[pl.ds(base, _BPW)], idx_v)
    bufs = (r0, r1)
    gsems = (g0, g1)
    wsems = (w0, w1)
    gcp = [None, None]
    wcp = [None, None]
    gcp[0] = pltpu.async_copy(
        table_hbm.at[idx_v.at[pl.ds(0, _CH)]], bufs[0], gsems[0])
    for c in range(_NCH):
        b = c % 2
        nb = (c + 1) % 2
        if c + 1 < _NCH:
            if wcp[nb] is not None:
                wcp[nb].wait()
            gcp[nb] = pltpu.async_copy(
                table_hbm.at[idx_v.at[pl.ds((c + 1) * _CH, _CH)]],
                bufs[nb], gsems[nb])
        gcp[b].wait()
        wcp[b] = pltpu.async_copy(
            bufs[b], out_hbm.at[pl.ds(base + c * _CH, _CH)], wsems[b])
    for w in wcp:
        w.wait()


def _sc_gather(table, idx):
    k = functools.partial(
        pl.kernel,
        mesh=plsc.VectorSubcoreMesh(
            core_axis_name="c", subcore_axis_name="s",
            num_cores=_NC, num_subcores=_NS),
        out_type=jax.ShapeDtypeStruct((Q, D), jnp.float32),
        scratch_types=[
            pltpu.VMEM((_BPW,), jnp.int32),
            pltpu.VMEM((_CH, D), jnp.float32),
            pltpu.VMEM((_CH, D), jnp.float32),
            pltpu.SemaphoreType.DMA,
            pltpu.SemaphoreType.DMA,
            pltpu.SemaphoreType.DMA,
            pltpu.SemaphoreType.DMA,
        ],
    )(_gather_body)
    return k(table, idx)


def kernel(source_features, target_features):
    source_norms = jnp.linalg.norm(source_features, axis=-1)
    matching_norms = jnp.linalg.norm(target_features, axis=-1)
    lm, li = _argmin_state(
        source_features, target_features,
        source_norms.reshape(Q, 1), matching_norms.reshape(1, T))
    idx = _finalize(lm, li).reshape(Q)
    return _sc_gather(target_features, idx)
